# baseline (speedup 1.0000x reference)
import jax
import jax.numpy as jnp
from jax import lax
from jax.experimental import pallas as pl
from jax.experimental.pallas import tpu as pltpu
from jax.experimental.pallas import tpu_sc as plsc

_D = 50
_NC = 2
_NS = 16
_NW = _NC * _NS
_BC = 8


def _embed_body(idx_hbm, table_hbm, out_hbm, idx_v, outb,
                gsem0, gsem1, osem0, osem1):
    per_w, p = idx_v.shape
    cl = _BC * p
    nchunk = per_w // _BC
    wid = lax.axis_index("s") * _NC + lax.axis_index("c")
    base_b = wid * per_w
    pltpu.sync_copy(idx_hbm.at[pl.ds(base_b, per_w)], idx_v)

    gsems = (gsem0, gsem1)
    osems = (osem0, osem1)

    def fire_chunk(c, par):
        for i in range(_BC):
            bb = c * _BC + i
            v0 = idx_v[bb, pl.ds(0, 16)]
            v1 = idx_v[bb, pl.ds(p - 16, 16)]
            for k in range(16):
                pltpu.async_copy(table_hbm.at[v0[k]],
                                 outb.at[par, i, k], gsems[par])
            for k in range(16, p):
                pltpu.async_copy(table_hbm.at[v1[k - (p - 16)]],
                                 outb.at[par, i, k], gsems[par])

    def wait_chunk(par):
        for i in range(_BC):
            for j in range(p):
                pltpu.make_async_copy(
                    table_hbm.at[0], outb.at[par, i, j], gsems[par]).wait()

    def out_slice(c):
        return out_hbm.at[pl.ds(base_b + c * _BC, _BC)]

    def fire_out(c, par):
        pltpu.async_copy(outb.at[par], out_slice(c), osems[par])

    def wait_out(par):
        pltpu.make_async_copy(outb.at[par], out_slice(0), osems[par]).wait()

    fire_chunk(0, 0)

    @pl.loop(0, nchunk // 2)
    def _cc(cc):
        for par in range(2):
            c = cc * 2 + par
            nxt = 1 - par

            @pl.when(c + 1 < nchunk)
            def _():
                fire_chunk(c + 1, nxt)

            wait_chunk(par)

            @pl.when(c >= 2)
            def _():
                wait_out(par)

            fire_out(c, par)

    for par in range(2):
        wait_out(par)


def kernel(indices, table):
    b, p = indices.shape
    per_w = b // _NW
    idx = indices.astype(jnp.int32)
    mesh = plsc.VectorSubcoreMesh(core_axis_name="c", subcore_axis_name="s")
    out = pl.kernel(
        _embed_body,
        out_type=jax.ShapeDtypeStruct((b, p, _D), jnp.float32),
        mesh=mesh,
        scratch_types=[
            pltpu.VMEM((per_w, p), jnp.int32),
            pltpu.VMEM((2, _BC, p, _D), jnp.float32),
            pltpu.SemaphoreType.DMA,
            pltpu.SemaphoreType.DMA,
            pltpu.SemaphoreType.DMA,
            pltpu.SemaphoreType.DMA,
        ],
        compiler_params=pltpu.CompilerParams(
            use_tc_tiling_on_sc=True, needs_layout_passes=False
        ),
    )(idx, table)
    return out


# speedup vs baseline: 1.0025x; 1.0025x over previous
import jax
import jax.numpy as jnp
from jax import lax
from jax.experimental import pallas as pl
from jax.experimental.pallas import tpu as pltpu
from jax.experimental.pallas import tpu_sc as plsc

_D = 50
_NC = 2
_NS = 16
_NW = _NC * _NS
_BC = 4


def _embed_body(idx_hbm, table_hbm, out_hbm, idx_v, outb,
                gsem0, gsem1, osem0, osem1):
    per_w, p = idx_v.shape
    cl = _BC * p
    nchunk = per_w // _BC
    wid = lax.axis_index("s") * _NC + lax.axis_index("c")
    base_b = wid * per_w
    pltpu.sync_copy(idx_hbm.at[pl.ds(base_b, per_w)], idx_v)

    gsems = (gsem0, gsem1)
    osems = (osem0, osem1)

    def fire_chunk(c, par):
        for i in range(_BC):
            bb = c * _BC + i
            v0 = idx_v[bb, pl.ds(0, 16)]
            v1 = idx_v[bb, pl.ds(p - 16, 16)]
            for k in range(16):
                pltpu.async_copy(table_hbm.at[v0[k]],
                                 outb.at[par, i, k], gsems[par])
            for k in range(16, p):
                pltpu.async_copy(table_hbm.at[v1[k - (p - 16)]],
                                 outb.at[par, i, k], gsems[par])

    def wait_chunk(par):
        for i in range(_BC):
            for j in range(p):
                pltpu.make_async_copy(
                    table_hbm.at[0], outb.at[par, i, j], gsems[par]).wait()

    def out_slice(c):
        return out_hbm.at[pl.ds(base_b + c * _BC, _BC)]

    def fire_out(c, par):
        pltpu.async_copy(outb.at[par], out_slice(c), osems[par])

    def wait_out(par):
        pltpu.make_async_copy(outb.at[par], out_slice(0), osems[par]).wait()

    fire_chunk(0, 0)

    @pl.loop(0, nchunk // 2)
    def _cc(cc):
        for par in range(2):
            c = cc * 2 + par
            nxt = 1 - par

            @pl.when(c + 1 < nchunk)
            def _():
                fire_chunk(c + 1, nxt)

            wait_chunk(par)

            @pl.when(c >= 2)
            def _():
                wait_out(par)

            fire_out(c, par)

    for par in range(2):
        wait_out(par)


def kernel(indices, table):
    b, p = indices.shape
    per_w = b // _NW
    idx = indices.astype(jnp.int32)
    mesh = plsc.VectorSubcoreMesh(core_axis_name="c", subcore_axis_name="s")
    out = pl.kernel(
        _embed_body,
        out_type=jax.ShapeDtypeStruct((b, p, _D), jnp.float32),
        mesh=mesh,
        scratch_types=[
            pltpu.VMEM((per_w, p), jnp.int32),
            pltpu.VMEM((2, _BC, p, _D), jnp.float32),
            pltpu.SemaphoreType.DMA,
            pltpu.SemaphoreType.DMA,
            pltpu.SemaphoreType.DMA,
            pltpu.SemaphoreType.DMA,
        ],
        compiler_params=pltpu.CompilerParams(
            use_tc_tiling_on_sc=True, needs_layout_passes=False
        ),
    )(idx, table)
    return out
